# Initial kernel scaffold; baseline (speedup 1.0000x reference)
import functools

import jax
import jax.numpy as jnp
from jax import lax
from jax.experimental import pallas as pl
from jax.experimental.pallas import tpu as pltpu
from jax.experimental.pallas import tpu_sc as plsc

N = 2
C = 128
D = H = W = 48
DHW = D * H * W
P = 100000
L = 16
NC, NS = 2, 16
NW = NC * NS
CH = 32
NCHUNK = P // CH
ITERS = -(-NCHUNK // NW)


def _dim_terms(cv):
    g = 2.0 * cv / 47.0 - 1.0
    ip = ((g + 1.0) * 48.0 - 1.0) / 2.0
    i0 = (ip + 1.0).astype(jnp.int32) - 1
    f0 = i0.astype(jnp.float32)
    w1 = ip - f0
    w0 = (f0 + 1.0) - ip
    v0 = (i0 >= 0) & (i0 <= 47)
    v1 = (i0 >= -1) & (i0 <= 46)
    w0 = jnp.where(v0, w0, 0.0)
    w1 = jnp.where(v1, w1, 0.0)
    i0c = jnp.clip(i0, 0, 47)
    i1c = jnp.clip(i0 + 1, 0, 47)
    return i0c, i1c, w0, w1


def _sc_body(table, cx, cy, cz, out, xv, yv, zv, idx_v, w_v, rows_v, ostage, sem):
    wid = lax.axis_index("s") * NC + lax.axis_index("c")

    for b in range(N):
        @pl.loop(0, ITERS)
        def _chunk_loop(i):
            c = wid + NW * i

            @pl.when(c < NCHUNK)
            def _():
                base = c * CH
                pltpu.sync_copy(cx.at[b, pl.ds(base, CH)], xv)
                pltpu.sync_copy(cy.at[b, pl.ds(base, CH)], yv)
                pltpu.sync_copy(cz.at[b, pl.ds(base, CH)], zv)

                for g16 in range(CH // L):
                    sl = pl.ds(g16 * L, L)
                    x0, x1, wx0, wx1 = _dim_terms(xv[sl])
                    y0, y1, wy0, wy1 = _dim_terms(yv[sl])
                    z0, z1, wz0, wz1 = _dim_terms(zv[sl])
                    xs = (x0, x1)
                    ys = (y0, y1)
                    zs = (z0, z1)
                    wxs = (wx0, wx1)
                    wys = (wy0, wy1)
                    wzs = (wz0, wz1)
                    for k in range(8):
                        dz, dy, dx = (k >> 2) & 1, (k >> 1) & 1, k & 1
                        idx = (zs[dz] * H + ys[dy]) * W + xs[dx] + b * DHW
                        wgt = wzs[dz] * wys[dy] * wxs[dx]
                        row, col = divmod(k * CH + g16 * L, 4 * CH)
                        idx_v[row, pl.ds(col, L)] = idx
                        w_v[row, pl.ds(col, L)] = wgt

                cp0 = pltpu.async_copy(table.at[idx_v.at[0]], rows_v.at[0], sem)
                cp1 = pltpu.async_copy(table.at[idx_v.at[1]], rows_v.at[1], sem)
                cp0.wait()
                cp1.wait()

                @pl.loop(0, CH)
                def _point_loop(p):
                    accs = [jnp.zeros((L,), jnp.float32)] * (C // L)
                    for k in range(8):
                        row = k // 4
                        col = (k % 4) * CH + p
                        colv = jnp.full((L,), col, dtype=jnp.int32)
                        roww = jnp.full((L,), row, dtype=jnp.int32)
                        wk = plsc.load_gather(w_v, [roww, colv])
                        for g in range(C // L):
                            r = rows_v[row, col, pl.ds(g * L, L)]
                            accs[g] = accs[g] + wk * r
                    for g in range(C // L):
                        ostage[p, pl.ds(g * L, L)] = accs[g]

                pltpu.sync_copy(ostage, out.at[b, pl.ds(base, CH)])


@jax.jit
def _sc_interp(table, cx, cy, cz):
    mesh = plsc.VectorSubcoreMesh(
        core_axis_name="c", subcore_axis_name="s", num_cores=NC, num_subcores=NS
    )
    f = pl.kernel(
        _sc_body,
        out_type=jax.ShapeDtypeStruct((N, P, C), jnp.float32),
        mesh=mesh,
        scratch_types=[
            pltpu.VMEM((CH,), jnp.float32),
            pltpu.VMEM((CH,), jnp.float32),
            pltpu.VMEM((CH,), jnp.float32),
            pltpu.VMEM((2, 4 * CH), jnp.int32),
            pltpu.VMEM((2, 4 * CH), jnp.float32),
            pltpu.VMEM((2, 4 * CH, C), jnp.float32),
            pltpu.VMEM((CH, C), jnp.float32),
            pltpu.SemaphoreType.DMA,
        ],
    )
    return f(table, cx, cy, cz)


def kernel(encoder_outputs, graph_coords):
    n, ch, d, h, w = encoder_outputs.shape
    table = (
        encoder_outputs.reshape(n, ch, d * h * w)
        .transpose(0, 2, 1)
        .reshape(n * d * h * w, ch)
    )
    coords = graph_coords.reshape(n, -1, 3)
    cx = coords[..., 0]
    cy = coords[..., 1]
    cz = coords[..., 2]
    out = _sc_interp(table, cx, cy, cz)
    return out.reshape(n, 1, 1, P, ch)


# speedup vs baseline: 1.2610x; 1.2610x over previous
import functools

import jax
import jax.numpy as jnp
from jax import lax
from jax.experimental import pallas as pl
from jax.experimental.pallas import tpu as pltpu
from jax.experimental.pallas import tpu_sc as plsc

N = 2
C = 128
D = H = W = 48
DHW = D * H * W
P = 100000
L = 16
NC, NS = 2, 16
NW = NC * NS
CH = 32
NCHUNK = P // CH
ITERS = -(-NCHUNK // NW)


def _dim_terms(cv):
    g = 2.0 * cv / 47.0 - 1.0
    ip = ((g + 1.0) * 48.0 - 1.0) / 2.0
    i0 = (ip + 1.0).astype(jnp.int32) - 1
    f0 = i0.astype(jnp.float32)
    w1 = ip - f0
    w0 = (f0 + 1.0) - ip
    v0 = (i0 >= 0) & (i0 <= 47)
    v1 = (i0 >= -1) & (i0 <= 46)
    w0 = jnp.where(v0, w0, 0.0)
    w1 = jnp.where(v1, w1, 0.0)
    i0c = jnp.clip(i0, 0, 47)
    i1c = jnp.clip(i0 + 1, 0, 47)
    return i0c, i1c, w0, w1


def _sc_body(table, cx, cy, cz, out, xv, yv, zv, idx_v, w_v, rows_v, ostage, sem):
    wid = lax.axis_index("s") * NC + lax.axis_index("c")

    for b in range(N):
        @pl.loop(0, ITERS)
        def _chunk_loop(i):
            c = wid + NW * i

            @pl.when(c < NCHUNK)
            def _():
                base = c * CH
                pltpu.sync_copy(cx.at[b, pl.ds(base, CH)], xv)
                pltpu.sync_copy(cy.at[b, pl.ds(base, CH)], yv)
                pltpu.sync_copy(cz.at[b, pl.ds(base, CH)], zv)

                for g16 in range(CH // L):
                    sl = pl.ds(g16 * L, L)
                    x0, x1, wx0, wx1 = _dim_terms(xv[sl])
                    y0, y1, wy0, wy1 = _dim_terms(yv[sl])
                    z0, z1, wz0, wz1 = _dim_terms(zv[sl])
                    xs = (x0, x1)
                    ys = (y0, y1)
                    zs = (z0, z1)
                    wxs = (wx0, wx1)
                    wys = (wy0, wy1)
                    wzs = (wz0, wz1)
                    for k in range(8):
                        dz, dy, dx = (k >> 2) & 1, (k >> 1) & 1, k & 1
                        idx = (zs[dz] * H + ys[dy]) * W + xs[dx] + b * DHW
                        wgt = wzs[dz] * wys[dy] * wxs[dx]
                        sk = pl.ds(k * CH + g16 * L, L)
                        idx_v[sk] = idx
                        w_v[sk] = wgt

                cp0 = pltpu.async_copy(
                    table.at[idx_v.at[pl.ds(0, 128)]],
                    rows_v.at[pl.ds(0, 128)], sem)
                cp1 = pltpu.async_copy(
                    table.at[idx_v.at[pl.ds(128, 128)]],
                    rows_v.at[pl.ds(128, 128)], sem)
                cp0.wait()
                cp1.wait()

                @pl.loop(0, CH)
                def _point_loop(p):
                    accs = [jnp.zeros((L,), jnp.float32)] * (C // L)
                    for k in range(8):
                        q = k * CH + p
                        wv = w_v[pl.ds(q, L)]
                        wk = jnp.full((L,), wv[0], dtype=jnp.float32)
                        for g in range(C // L):
                            r = rows_v[q, pl.ds(g * L, L)]
                            accs[g] = accs[g] + wk * r
                    for g in range(C // L):
                        ostage[p, pl.ds(g * L, L)] = accs[g]

                pltpu.sync_copy(ostage, out.at[b, pl.ds(base, CH)])


@jax.jit
def _sc_interp(table, cx, cy, cz):
    mesh = plsc.VectorSubcoreMesh(
        core_axis_name="c", subcore_axis_name="s", num_cores=NC, num_subcores=NS
    )
    f = pl.kernel(
        _sc_body,
        out_type=jax.ShapeDtypeStruct((N, P, C), jnp.float32),
        mesh=mesh,
        scratch_types=[
            pltpu.VMEM((CH,), jnp.float32),
            pltpu.VMEM((CH,), jnp.float32),
            pltpu.VMEM((CH,), jnp.float32),
            pltpu.VMEM((8 * CH,), jnp.int32),
            pltpu.VMEM((8 * CH + L,), jnp.float32),
            pltpu.VMEM((8 * CH, C), jnp.float32),
            pltpu.VMEM((CH, C), jnp.float32),
            pltpu.SemaphoreType.DMA,
        ],
    )
    return f(table, cx, cy, cz)


def kernel(encoder_outputs, graph_coords):
    n, ch, d, h, w = encoder_outputs.shape
    table = (
        encoder_outputs.reshape(n, ch, d * h * w)
        .transpose(0, 2, 1)
        .reshape(n * d * h * w, ch)
    )
    coords = graph_coords.reshape(n, -1, 3)
    cx = coords[..., 0]
    cy = coords[..., 1]
    cz = coords[..., 2]
    out = _sc_interp(table, cx, cy, cz)
    return out.reshape(n, 1, 1, P, ch)
